# baseline (speedup 1.0000x reference)
import numpy as np
import jax
import jax.numpy as jnp
from jax.experimental import pallas as pl
from jax.experimental.pallas import tpu as pltpu

_MP = 7
_MP_PAD = 2
_CK = 3
_SIGMA = 3.0

_k1d = np.exp(-(np.arange(_CK, dtype=np.float32) - _CK // 2) ** 2
              / (2.0 * np.float32(_SIGMA) ** 2)).astype(np.float32)
_k1d = (_k1d / _k1d.sum()).astype(np.float32)
_K0, _K1, _K2 = (float(v) for v in _k1d)


def _grid2image_block(x_ref, o_ref):
    bb, depth, H, W = x_ref.shape
    Hp, Wp = H + 2 * _MP_PAD, W + 2 * _MP_PAD
    Ho, Wo = Hp - _MP + 1, Wp - _MP + 1

    xb = x_ref[...]

    neg_h = jnp.full((bb, depth, _MP_PAD, W), -jnp.inf, dtype=jnp.float32)
    ph = jnp.concatenate([neg_h, xb, neg_h], axis=2)
    t = jnp.maximum(ph[:, :, 0:Hp - 1], ph[:, :, 1:Hp])
    t = jnp.maximum(t[:, :, 0:Hp - 3], t[:, :, 2:Hp - 1])
    rh = jnp.maximum(t[:, :, 0:Ho], t[:, :, 3:Ho + 3])

    neg_w = jnp.full((bb, depth, Ho, _MP_PAD), -jnp.inf, dtype=jnp.float32)
    pw = jnp.concatenate([neg_w, rh, neg_w], axis=3)
    s = jnp.maximum(pw[..., 0:Wp - 1], pw[..., 1:Wp])
    s = jnp.maximum(s[..., 0:Wp - 3], s[..., 2:Wp - 1])
    p = jnp.maximum(s[..., 0:Wo], s[..., 3:Wo + 3])

    zh = jnp.zeros((bb, depth, 1, Wo), jnp.float32)
    ch = jnp.concatenate([zh, p, zh], axis=2)
    g = (_K0 * ch[:, :, 0:Ho] + _K1 * ch[:, :, 1:Ho + 1]
         + _K2 * ch[:, :, 2:Ho + 2])

    zw = jnp.zeros((bb, depth, Ho, 1), jnp.float32)
    cw = jnp.concatenate([zw, g, zw], axis=3)
    cv = (_K0 * cw[..., 0:Wo] + _K1 * cw[..., 1:Wo + 1]
          + _K2 * cw[..., 2:Wo + 2])

    acc = jnp.max(cv, axis=1)
    m = jnp.max(acc, axis=(1, 2), keepdims=True)
    out = 1.0 - acc * (1.0 / m)
    o_ref[...] = jnp.broadcast_to(out[:, None], (bb, 3, Ho, Wo))


def kernel(x):
    x = x.astype(jnp.float32)
    B, D, H, W = x.shape
    Ho, Wo = H - 2, W - 2

    bb = 2
    while B % bb:
        bb //= 2

    return pl.pallas_call(
        _grid2image_block,
        out_shape=jax.ShapeDtypeStruct((B, 3, Ho, Wo), jnp.float32),
        grid=(B // bb,),
        in_specs=[pl.BlockSpec((bb, D, H, W), lambda i: (i, 0, 0, 0))],
        out_specs=pl.BlockSpec((bb, 3, Ho, Wo), lambda i: (i, 0, 0, 0)),
        compiler_params=pltpu.CompilerParams(
            dimension_semantics=("parallel",)),
    )(x)


# speedup vs baseline: 1.0305x; 1.0305x over previous
import numpy as np
import jax
import jax.numpy as jnp
from jax.experimental import pallas as pl
from jax.experimental.pallas import tpu as pltpu

_MP = 7
_MP_PAD = 2
_CK = 3
_SIGMA = 3.0

_k1d = np.exp(-(np.arange(_CK, dtype=np.float32) - _CK // 2) ** 2
              / (2.0 * np.float32(_SIGMA) ** 2)).astype(np.float32)
_k1d = (_k1d / _k1d.sum()).astype(np.float32)
_K0, _K1, _K2 = (float(v) for v in _k1d)


def _grid2image_block(x_ref, o_ref):
    bb, depth, H, W = x_ref.shape
    Hp, Wp = H + 2 * _MP_PAD, W + 2 * _MP_PAD
    Ho, Wo = Hp - _MP + 1, Wp - _MP + 1

    xb = x_ref[...]

    neg_h = jnp.full((bb, depth, _MP_PAD, W), -jnp.inf, dtype=jnp.float32)
    ph = jnp.concatenate([neg_h, xb, neg_h], axis=2)
    t = jnp.maximum(ph[:, :, 0:Hp - 1], ph[:, :, 1:Hp])
    t = jnp.maximum(t[:, :, 0:Hp - 3], t[:, :, 2:Hp - 1])
    rh = jnp.maximum(t[:, :, 0:Ho], t[:, :, 3:Ho + 3])

    neg_w = jnp.full((bb, depth, Ho, _MP_PAD), -jnp.inf, dtype=jnp.float32)
    pw = jnp.concatenate([neg_w, rh, neg_w], axis=3)
    s = jnp.maximum(pw[..., 0:Wp - 1], pw[..., 1:Wp])
    s = jnp.maximum(s[..., 0:Wp - 3], s[..., 2:Wp - 1])
    p = jnp.maximum(s[..., 0:Wo], s[..., 3:Wo + 3])

    zh = jnp.zeros((bb, depth, 1, Wo), jnp.float32)
    ch = jnp.concatenate([zh, p, zh], axis=2)
    g = (_K0 * ch[:, :, 0:Ho] + _K1 * ch[:, :, 1:Ho + 1]
         + _K2 * ch[:, :, 2:Ho + 2])

    zw = jnp.zeros((bb, depth, Ho, 1), jnp.float32)
    cw = jnp.concatenate([zw, g, zw], axis=3)
    cv = (_K0 * cw[..., 0:Wo] + _K1 * cw[..., 1:Wo + 1]
          + _K2 * cw[..., 2:Wo + 2])

    acc = jnp.max(cv, axis=1)
    m = jnp.max(acc, axis=(1, 2), keepdims=True)
    out = 1.0 - acc * (1.0 / m)
    o_ref[...] = jnp.broadcast_to(out[:, None], (bb, 3, Ho, Wo))


def kernel(x):
    x = x.astype(jnp.float32)
    B, D, H, W = x.shape
    Ho, Wo = H - 2, W - 2

    bb = 8
    while B % bb:
        bb //= 2

    return pl.pallas_call(
        _grid2image_block,
        out_shape=jax.ShapeDtypeStruct((B, 3, Ho, Wo), jnp.float32),
        grid=(B // bb,),
        in_specs=[pl.BlockSpec((bb, D, H, W), lambda i: (i, 0, 0, 0))],
        out_specs=pl.BlockSpec((bb, 3, Ho, Wo), lambda i: (i, 0, 0, 0)),
        compiler_params=pltpu.CompilerParams(
            dimension_semantics=("parallel",)),
    )(x)
